# baseline (speedup 1.0000x reference)
import functools

import jax
import jax.numpy as jnp
from jax import lax
from jax.experimental import pallas as pl
from jax.experimental.pallas import tpu as pltpu
from jax.experimental.pallas import tpu_sc as plsc

D = 64
NC = 2
NS = 16
NW = NC * NS
CHUNK = 512
G = 1
GROUP = G * CHUNK

VOCAB_FULL = 1000000
NBLK = VOCAB_FULL // 128
VTAIL = VOCAB_FULL - NBLK * 128
T2_ROWS = VOCAB_FULL // 2


def _reformat_body(tab_t, tail, t2, ibuf0, ibuf1, obuf0, obuf1, tbuf,
                   gsem0, gsem1, ssem0, ssem1):
    wid = lax.axis_index("s") * NC + lax.axis_index("c")

    base = NBLK // NW
    extra = NBLK - base * NW
    start = wid * base + jnp.minimum(wid, extra)
    count = base + jnp.where(wid < extra, 1, 0)

    lanes = lax.iota(jnp.int32, 16)
    ibufs = (ibuf0, ibuf1)
    obufs = (obuf0, obuf1)
    gsems = (gsem0, gsem1)
    ssems = (ssem0, ssem1)

    dvecs = []
    tadds = []
    for k in range(8):
        c = 16 * k + lanes
        dvecs.append(lax.rem(c, D))
        tadds.append(jnp.where(c >= D, 1, 0).astype(jnp.int32))

    def issue_load(slot, j):
        pltpu.async_copy(tab_t.at[:, pl.ds(j * 128, 128)], ibufs[slot],
                         gsems[slot])

    def wait_load(slot, j):
        pltpu.make_async_copy(tab_t.at[:, pl.ds(j * 128, 128)], ibufs[slot],
                              gsems[slot]).wait()

    def issue_store(slot, j):
        pltpu.async_copy(obufs[slot], t2.at[pl.ds(j * D, D)], ssems[slot])

    def wait_store(slot, j):
        pltpu.make_async_copy(obufs[slot], t2.at[pl.ds(j * D, D)],
                              ssems[slot]).wait()

    def transpose_block(ibuf, obuf, nrows):
        def row(r, carry):
            v = (2 * r).astype(jnp.int32)
            for k in range(8):
                vec = plsc.load_gather(ibuf, [dvecs[k], tadds[k] + v])
                obuf[r, pl.ds(16 * k, 16)] = vec
            return carry
        lax.fori_loop(0, nrows, row, 0)

    issue_load(0, start)

    def body(i, carry):
        j = start + i

        def arm(s):
            wait_load(s, j)

            @pl.when(i + 1 < count)
            def _():
                issue_load(1 - s, j + 1)

            @pl.when(i >= 2)
            def _():
                wait_store(s, j - 2)

            transpose_block(ibufs[s], obufs[s], D)
            issue_store(s, j)

        @pl.when(lax.rem(i, 2) == 0)
        def _():
            arm(0)

        @pl.when(lax.rem(i, 2) == 1)
        def _():
            arm(1)

        return carry

    lax.fori_loop(0, count, body, 0)
    wait_store(0, 0)
    wait_store(1, 0)

    @pl.when(wid == NW - 1)
    def _():
        pltpu.sync_copy(tail, tbuf)

        def row(r, carry):
            v = (2 * r).astype(jnp.int32)
            for k in range(8):
                vec = plsc.load_gather(tbuf, [tadds[k] + v, dvecs[k]])
                obuf0[r, pl.ds(16 * k, 16)] = vec
            return carry
        lax.fori_loop(0, VTAIL // 2, row, 0)
        pltpu.sync_copy(obuf0.at[pl.ds(0, VTAIL // 2)],
                        t2.at[pl.ds(NBLK * D, VTAIL // 2)])


def _make_pair_table(table):
    tab_t = table.T
    tail = table[NBLK * 128:, :]
    mesh = plsc.VectorSubcoreMesh(core_axis_name="c", subcore_axis_name="s")
    run = pl.kernel(
        _reformat_body,
        out_type=jax.ShapeDtypeStruct((T2_ROWS, 128), jnp.float32),
        mesh=mesh,
        scratch_types=[
            pltpu.VMEM((D, 128), jnp.float32),
            pltpu.VMEM((D, 128), jnp.float32),
            pltpu.VMEM((D, 128), jnp.float32),
            pltpu.VMEM((D, 128), jnp.float32),
            pltpu.VMEM((D, D), jnp.float32),
            pltpu.SemaphoreType.DMA,
            pltpu.SemaphoreType.DMA,
            pltpu.SemaphoreType.DMA,
            pltpu.SemaphoreType.DMA,
        ],
        compiler_params=pltpu.CompilerParams(use_tc_tiling_on_sc=True,
                                             needs_layout_passes=False),
    )
    return run(tab_t, tail)


def _gather_body(n_chunks, idx_hbm, table_hbm, out_hbm,
                 idx_v, buf_p, buf_q, gsem_p, gsem_q, ssem_p, ssem_q):
    wid = lax.axis_index("s") * NC + lax.axis_index("c")
    base = wid * (n_chunks * CHUNK)
    n_pairs = n_chunks // (2 * G)

    pltpu.sync_copy(idx_hbm.at[wid], idx_v)

    def issue_gathers(buf, sem, g):
        for b in range(G):
            pltpu.async_copy(table_hbm.at[idx_v.at[g * G + b]],
                             buf.at[pl.ds(b * CHUNK, CHUNK)], sem)

    def wait_gathers(buf, sem, g):
        for b in range(G):
            pltpu.make_async_copy(table_hbm.at[idx_v.at[g * G + b]],
                                  buf.at[pl.ds(b * CHUNK, CHUNK)], sem).wait()

    def issue_store(buf, sem, g):
        pltpu.async_copy(buf, out_hbm.at[pl.ds(base + g * GROUP, GROUP)], sem)

    def wait_store(buf, sem, g):
        pltpu.make_async_copy(buf, out_hbm.at[pl.ds(base + g * GROUP, GROUP)],
                              sem).wait()

    issue_gathers(buf_p, gsem_p, 0)

    def outer(i, carry):
        g_p = 2 * i
        g_q = 2 * i + 1

        wait_gathers(buf_p, gsem_p, g_p)
        issue_store(buf_p, ssem_p, g_p)

        @pl.when(i > 0)
        def _():
            wait_store(buf_q, ssem_q, g_q - 2)

        issue_gathers(buf_q, gsem_q, g_q)
        wait_gathers(buf_q, gsem_q, g_q)
        issue_store(buf_q, ssem_q, g_q)
        wait_store(buf_p, ssem_p, g_p)

        @pl.when(i + 1 < n_pairs)
        def _():
            issue_gathers(buf_p, gsem_p, g_p + 2)

        return carry

    lax.fori_loop(0, n_pairs, outer, 0)
    wait_store(buf_q, ssem_q, 2 * n_pairs - 1)


def kernel(x, table):
    b, t = x.shape
    n = b * t
    assert n % (NW * 2 * GROUP) == 0
    n_chunks = n // (NW * CHUNK)

    t2 = _make_pair_table(table)
    table_rm = t2.reshape(T2_ROWS * 128).reshape(VOCAB_FULL, D)

    idx = x.reshape(NW, n_chunks, CHUNK).astype(jnp.int32)

    mesh = plsc.VectorSubcoreMesh(core_axis_name="c", subcore_axis_name="s")
    run = pl.kernel(
        functools.partial(_gather_body, n_chunks),
        out_type=jax.ShapeDtypeStruct((n, D), jnp.float32),
        mesh=mesh,
        scratch_types=[
            pltpu.VMEM((n_chunks, CHUNK), jnp.int32),
            pltpu.VMEM((CHUNK, D), jnp.float32),
            pltpu.VMEM((CHUNK, D), jnp.float32),
            pltpu.SemaphoreType.DMA,
            pltpu.SemaphoreType.DMA,
            pltpu.SemaphoreType.DMA,
            pltpu.SemaphoreType.DMA,
        ],
        compiler_params=pltpu.CompilerParams(use_tc_tiling_on_sc=False),
    )
    out = run(idx, table_rm)
    return out.reshape(b, t, D)


# speedup vs baseline: 1.3319x; 1.3319x over previous
import jax
import jax.numpy as jnp
from jax import lax
from jax.experimental import pallas as pl
from jax.experimental.pallas import tpu as pltpu
from jax.experimental.pallas import tpu_sc as plsc

D = 64
NC = 2
NS = 16
NW = NC * NS
BB = 128
T_STEPS = 200
T2_ROWS = 500000


def _body(x_t, t2, out, xv, rb0, rb1, gb0, gb1, ob0, ob1,
          gsem0, gsem1, ssem0, ssem1):
    wid = lax.axis_index("s") * NC + lax.axis_index("c")
    col = wid * BB

    lanes = lax.iota(jnp.int32, 16)
    rbs = (rb0, rb1)
    gbs = (gb0, gb1)
    obs = (ob0, ob1)
    gsems = (gsem0, gsem1)
    ssems = (ssem0, ssem1)

    pltpu.sync_copy(x_t.at[:, pl.ds(col, BB)], xv)

    def prep(slot, t):
        for g in range(8):
            idx = xv[t, pl.ds(16 * g, 16)]
            rbs[slot][pl.ds(16 * g, 16)] = lax.shift_right_logical(idx, 1)

    def issue_gather(slot, t):
        pltpu.async_copy(t2.at[rbs[slot]], gbs[slot], gsems[slot])

    def wait_gather(slot):
        pltpu.make_async_copy(t2.at[rbs[slot]], gbs[slot], gsems[slot]).wait()

    def issue_store(slot, t):
        pltpu.async_copy(obs[slot], out.at[t, :, pl.ds(col, BB)], ssems[slot])

    def wait_store(slot, t):
        pltpu.make_async_copy(obs[slot], out.at[t, :, pl.ds(col, BB)],
                              ssems[slot]).wait()

    def transpose(slot, t):
        gb = gbs[slot]
        ob = obs[slot]
        bvecs = []
        cvecs = []
        for g in range(8):
            idx = xv[t, pl.ds(16 * g, 16)]
            par = lax.bitwise_and(idx, 1)
            bvecs.append((16 * g) + lanes)
            cvecs.append(lax.shift_left(par, 6))

        def drow(d, carry):
            vecs = [plsc.load_gather(gb, [bvecs[g], cvecs[g] + d])
                    for g in range(8)]
            for g in range(8):
                ob[d, pl.ds(16 * g, 16)] = vecs[g]
            return carry

        lax.fori_loop(0, D, drow, 0, unroll=2)

    prep(0, 0)
    issue_gather(0, 0)

    def step(t, carry):
        def arm(s):
            wait_gather(s)

            @pl.when(t + 1 < T_STEPS)
            def _():
                prep(1 - s, t + 1)
                issue_gather(1 - s, t + 1)

            @pl.when(t >= 2)
            def _():
                wait_store(s, t - 2)

            transpose(s, t)
            issue_store(s, t)

        @pl.when(lax.rem(t, 2) == 0)
        def _():
            arm(0)

        @pl.when(lax.rem(t, 2) == 1)
        def _():
            arm(1)

        return carry

    lax.fori_loop(0, T_STEPS, step, 0)
    wait_store(0, 0)
    wait_store(1, 0)


def kernel(x, table):
    b, t = x.shape
    assert (b, t) == (4096, T_STEPS) and table.shape == (1000000, D)

    x_t = x.T
    t2 = table.reshape(T2_ROWS, 128)

    mesh = plsc.VectorSubcoreMesh(core_axis_name="c", subcore_axis_name="s")
    run = pl.kernel(
        _body,
        out_type=jax.ShapeDtypeStruct((T_STEPS, D, b), jnp.float32),
        mesh=mesh,
        scratch_types=[
            pltpu.VMEM((T_STEPS, BB), jnp.int32),
            pltpu.VMEM((BB,), jnp.int32),
            pltpu.VMEM((BB,), jnp.int32),
            pltpu.VMEM((BB, 128), jnp.float32),
            pltpu.VMEM((BB, 128), jnp.float32),
            pltpu.VMEM((D, BB), jnp.float32),
            pltpu.VMEM((D, BB), jnp.float32),
            pltpu.SemaphoreType.DMA,
            pltpu.SemaphoreType.DMA,
            pltpu.SemaphoreType.DMA,
            pltpu.SemaphoreType.DMA,
        ],
        compiler_params=pltpu.CompilerParams(use_tc_tiling_on_sc=True,
                                             needs_layout_passes=False),
    )
    out3 = run(x_t, t2)
    return out3.transpose(2, 0, 1)


# speedup vs baseline: 1.3881x; 1.0422x over previous
import jax
import jax.numpy as jnp
from jax import lax
from jax.experimental import pallas as pl
from jax.experimental.pallas import tpu as pltpu
from jax.experimental.pallas import tpu_sc as plsc

D = 64
NC = 2
NS = 16
NW = NC * NS
BB = 128
T_STEPS = 200
GT = 2


def _body(x_t, t2p, out5, xv, gb0, gb1, ob0, ob1,
          gsem0, gsem1, ssem0, ssem1):
    wid = lax.axis_index("s") * NC + lax.axis_index("c")
    col = wid * BB

    lanes = lax.iota(jnp.int32, 16)
    zero16 = lanes * 0
    gbs = (gb0, gb1)
    obs = (ob0, ob1)
    gsems = (gsem0, gsem1)
    ssems = (ssem0, ssem1)

    pltpu.sync_copy(x_t.at[:, pl.ds(col, BB)], xv)

    def issue_gather(slot, i):
        for tt in range(GT):
            pltpu.async_copy(t2p.at[xv.at[GT * i + tt]], gbs[slot].at[tt],
                             gsems[slot])

    def wait_gather(slot, i):
        for tt in range(GT):
            pltpu.make_async_copy(t2p.at[xv.at[GT * i + tt]],
                                  gbs[slot].at[tt], gsems[slot]).wait()

    def issue_store(slot, t):
        pltpu.async_copy(obs[slot], out5.at[t, :, wid], ssems[slot])

    def wait_store(slot, t):
        pltpu.make_async_copy(obs[slot], out5.at[t, :, wid],
                              ssems[slot]).wait()

    def transpose(gslot, tt, oslot):
        gb = gbs[gslot]
        ob = obs[oslot]
        ttv = zero16 + tt

        def drow(d, carry):
            cv = zero16 + d
            vecs = [plsc.load_gather(gb, [ttv, (16 * g) + lanes, cv])
                    for g in range(8)]
            k = lax.div(d, 8)
            r = lax.rem(d, 8)
            for g in range(8):
                ob[k, r, pl.ds(16 * g, 16)] = vecs[g]
            return carry

        lax.fori_loop(0, D, drow, 0, unroll=2)

    n_pairs = T_STEPS // GT
    issue_gather(0, 0)

    def step(i, carry):
        def arm(s):
            wait_gather(s, i)

            @pl.when(i + 1 < n_pairs)
            def _():
                issue_gather(1 - s, i + 1)

            for tt in range(GT):
                t = GT * i + tt

                @pl.when(i >= 1)
                def _():
                    wait_store(tt, t - GT)

                transpose(s, tt, tt)
                issue_store(tt, t)

        @pl.when(lax.rem(i, 2) == 0)
        def _():
            arm(0)

        @pl.when(lax.rem(i, 2) == 1)
        def _():
            arm(1)

        return carry

    lax.fori_loop(0, n_pairs, step, 0)
    wait_store(0, 0)
    wait_store(1, 1)


def kernel(x, table):
    b, t = x.shape
    assert (b, t) == (4096, T_STEPS) and table.shape == (1000000, D)

    x_t = x.T
    t2p = jnp.pad(table, ((0, 0), (0, 64)))

    mesh = plsc.VectorSubcoreMesh(core_axis_name="c", subcore_axis_name="s")
    run = pl.kernel(
        _body,
        out_type=jax.ShapeDtypeStruct((T_STEPS, 8, NW, 8, BB), jnp.float32),
        mesh=mesh,
        scratch_types=(
            [pltpu.VMEM((T_STEPS, BB), jnp.int32)]
            + [pltpu.VMEM((GT, BB, 128), jnp.float32) for _ in range(2)]
            + [pltpu.VMEM((8, 8, BB), jnp.float32) for _ in range(2)]
            + [pltpu.SemaphoreType.DMA for _ in range(4)]
        ),
        compiler_params=pltpu.CompilerParams(needs_layout_passes=False),
    )
    out5 = run(x_t, t2p)
    return out5.transpose(2, 4, 0, 1, 3).reshape(b, t, D)
